# Initial kernel scaffold; baseline (speedup 1.0000x reference)
import jax
import jax.numpy as jnp
from jax.experimental import pallas as pl


def kernel(predict, target):
    raise NotImplementedError("write your pallas kernel here")


# speedup vs baseline: 17.6911x; 17.6911x over previous
import functools

import jax
import jax.numpy as jnp
from jax import lax
from jax.experimental import pallas as pl
from jax.experimental.pallas import tpu as pltpu
from jax.experimental.pallas import tpu_sc as plsc

_N = 20
_P = _N ** 3
_L = 16
_INF = float("inf")

@functools.lru_cache(maxsize=1)
def _build_sc_kernel():
    mesh = plsc.VectorSubcoreMesh(
        core_axis_name="c", subcore_axis_name="s", num_cores=2, num_subcores=16
    )
    return functools.partial(
        pl.kernel,
        out_type=jax.ShapeDtypeStruct((4, _L), jnp.float32),
        mesh=mesh,
        scratch_types=[
            pltpu.VMEM((_P,), jnp.float32),
            pltpu.VMEM((_P,), jnp.float32),
            pltpu.VMEM((_P,), jnp.float32),
            pltpu.VMEM((_P,), jnp.float32),
            pltpu.VMEM((_P,), jnp.float32),
            pltpu.VMEM((_L,), jnp.float32),
            pltpu.VMEM((_L,), jnp.float32),
            pltpu.VMEM((_L,), jnp.float32),
        ],
        compiler_params=pltpu.CompilerParams(needs_layout_passes=False),
    )(_hausdorff_sc_body)


def _hausdorff_sc_body(pred_hbm, targ_hbm, out_hbm,
                       pred_v, targ_v, q_v, da_v, db_v, osrc_v, acc_v, res_v):
    c = lax.axis_index("c")
    s = lax.axis_index("s")
    vol = c * 2 + s

    @pl.when(s < 2)
    def _body():
        batch = vol // 2
        lane = lax.iota(jnp.int32, _L)
        is_a = jnp.broadcast_to((vol % 2) == 0, (_L,))

        pltpu.sync_copy(pred_hbm.at[batch], pred_v)
        pltpu.sync_copy(targ_hbm.at[batch], targ_v)

        osrc_v[...] = jnp.zeros((_L,), jnp.float32)
        acc_v[...] = jnp.full((_L,), -1.0, jnp.float32)

        def init_body(j, carry):
            sl = pl.ds(j * _L, _L)
            a = pred_v[sl] > 0.5
            b = targ_v[sl] > 0.5
            qa = jnp.where(a & (~b), 1.0, 0.0)
            qb = jnp.where(b & (~a), 1.0, 0.0)
            af = jnp.where(a, 1.0, 0.0)
            bf = jnp.where(b, 1.0, 0.0)
            src = jnp.where(is_a, bf, af)
            q_v[sl] = jnp.where(is_a, qa, qb)
            da_v[sl] = jnp.where(src > 0.5, 0.0, _INF)
            osrc_v[...] = jnp.maximum(osrc_v[...], src)
            return carry

        lax.fori_loop(0, _P // _L, init_body, 0)

        w = [[float((z - zp) ** 2) for z in range(_N)] for zp in range(_N)]

        def run_pass(src_ref, dst_ref, stride, base_of, last):
            def group_body(g, carry):
                line = g * _L + lane
                base = base_of(line)
                ins = [plsc.load_gather(src_ref, [base + stride * zp])
                       for zp in range(_N)]
                for z in range(_N):
                    o = ins[0] + w[0][z]
                    for zp in range(1, _N):
                        o = jnp.minimum(o, ins[zp] + w[zp][z])
                    if last:
                        qv = plsc.load_gather(q_v, [base + stride * z])
                        acc_v[...] = jnp.maximum(
                            acc_v[...], jnp.where(qv > 0.5, o, -1.0))
                    else:
                        plsc.store_scatter(dst_ref, [base + stride * z], o)
                return carry

            lax.fori_loop(0, (_N * _N) // _L, group_body, 0)

        run_pass(da_v, db_v, 1, lambda l: l * _N, False)
        run_pass(db_v, da_v, _N,
                 lambda l: (l // _N) * (_N * _N) + (l % _N), False)
        run_pass(da_v, db_v, _N * _N, lambda l: l, True)

        mx = jnp.max(acc_v[...])
        osrc = jnp.max(osrc_v[...])
        res_v[...] = jnp.where(lane == 0, mx,
                               jnp.where(lane == 1, osrc, 0.0))
        pltpu.sync_copy(res_v, out_hbm.at[vol])


def kernel(predict, target):
    pred = predict.reshape(2, _P)
    targ = target.reshape(2, _P)
    out = _build_sc_kernel()(pred, targ)
    maxq = out[:, 0]
    osrc = out[:, 1]
    d = jnp.sqrt(jnp.maximum(maxq, 0.0)) / jnp.float32(_N)
    dist_a = jnp.where(maxq[0::2] < 0.0, 0.0, d[0::2])
    dist_b = jnp.where(maxq[1::2] < 0.0, 0.0,
                       jnp.where(osrc[1::2] > 0.0, d[1::2], 999.0))
    return jnp.mean(jnp.maximum(dist_a, dist_b))


# speedup vs baseline: 25.7757x; 1.4570x over previous
import functools

import jax
import jax.numpy as jnp
from jax import lax
from jax.experimental import pallas as pl
from jax.experimental.pallas import tpu as pltpu
from jax.experimental.pallas import tpu_sc as plsc

_N = 20
_P = _N ** 3
_L = 16
_T = 5
_LINES = 400 // _T
_G = _LINES // _L
_SLAB = _P // _T
_INF = float("inf")


@functools.lru_cache(maxsize=1)
def _build_sc_kernel():
    mesh = plsc.VectorSubcoreMesh(
        core_axis_name="c", subcore_axis_name="s", num_cores=2, num_subcores=16
    )
    return functools.partial(
        pl.kernel,
        out_type=jax.ShapeDtypeStruct((4, _T, _L), jnp.float32),
        mesh=mesh,
        scratch_types=[
            pltpu.VMEM((_P,), jnp.float32),
            pltpu.VMEM((_P,), jnp.float32),
            pltpu.VMEM((_P,), jnp.float32),
            pltpu.VMEM((_P,), jnp.float32),
            pltpu.VMEM((_L,), jnp.float32),
            pltpu.VMEM((_L,), jnp.float32),
            pltpu.VMEM_SHARED((2 * _P,), jnp.float32),
        ],
        compiler_params=pltpu.CompilerParams(needs_layout_passes=False),
    )(_hausdorff_sc_body)


def _hausdorff_sc_body(pred_hbm, targ_hbm, out_hbm,
                       src_v, oth_v, da_v, db_v, acc_v, res_v, shared):
    c = lax.axis_index("c")
    s = lax.axis_index("s")
    active = s < 2 * _T
    vloc = s // _T
    vol = c * 2 + vloc
    t = s % _T
    batch = vol // 2
    is_a = (vol % 2) == 0
    lane = lax.iota(jnp.int32, _L)
    w = [[float((z - zp) ** 2) for z in range(_N)] for zp in range(_N)]

    def run_pass(load_in, handle_out, base_of, stride):
        def group_body(g, carry):
            line = t * _LINES + g * _L + lane
            base = base_of(line)
            ins = [load_in(base + stride * zp) for zp in range(_N)]
            for z in range(_N):
                o = ins[0] + w[0][z]
                for zp in range(1, _N):
                    o = jnp.minimum(o, ins[zp] + w[zp][z])
                handle_out(base + stride * z, o)
            return carry
        lax.fori_loop(0, _G, group_body, 0)

    @pl.when(active)
    def _phase1():
        @pl.when(is_a)
        def _():
            pltpu.sync_copy(targ_hbm.at[batch], src_v)
            pltpu.sync_copy(pred_hbm.at[batch], oth_v)

        @pl.when(jnp.logical_not(is_a))
        def _():
            pltpu.sync_copy(pred_hbm.at[batch], src_v)
            pltpu.sync_copy(targ_hbm.at[batch], oth_v)

        def load_z(idx):
            return jnp.where(plsc.load_gather(src_v, [idx]) > 0.5, 0.0, _INF)

        run_pass(load_z,
                 lambda idx, o: plsc.store_scatter(da_v, [idx], o),
                 lambda l: l * _N, 1)

        run_pass(lambda idx: plsc.load_gather(da_v, [idx]),
                 lambda idx, o: plsc.store_scatter(db_v, [idx], o),
                 lambda l: (l // _N) * (_N * _N) + (l % _N), _N)

        pltpu.sync_copy(db_v.at[pl.ds(t * _SLAB, _SLAB)],
                        shared.at[pl.ds(vloc * _P + t * _SLAB, _SLAB)])

    plsc.subcore_barrier()

    @pl.when(active)
    def _phase2():
        pltpu.sync_copy(shared.at[pl.ds(vloc * _P, _P)], da_v)
        acc_v[...] = jnp.full((_L,), -1.0, jnp.float32)

        def reduce_x(idx, o):
            qs = plsc.load_gather(src_v, [idx])
            qo = plsc.load_gather(oth_v, [idx])
            q = (qo > 0.5) & jnp.logical_not(qs > 0.5)
            acc_v[...] = jnp.maximum(acc_v[...], jnp.where(q, o, -1.0))

        run_pass(lambda idx: plsc.load_gather(da_v, [idx]),
                 reduce_x, lambda l: l, _N * _N)

        mx = jnp.max(acc_v[...])
        res_v[...] = jnp.where(lane == 0, mx, -1.0)
        pltpu.sync_copy(res_v, out_hbm.at[vol, t])


def kernel(predict, target):
    pred = predict.reshape(2, _P)
    targ = target.reshape(2, _P)
    out = _build_sc_kernel()(pred, targ)
    maxq = jnp.max(out, axis=(1, 2))
    d = jnp.sqrt(jnp.maximum(maxq, 0.0)) / jnp.float32(_N)
    dist_a = jnp.where(maxq[0::2] < 0.0, 0.0, d[0::2])
    dist_b = jnp.where(maxq[1::2] < 0.0, 0.0,
                       jnp.where(maxq[1::2] > 1e9, 999.0, d[1::2]))
    return jnp.mean(jnp.maximum(dist_a, dist_b))


# speedup vs baseline: 26.1963x; 1.0163x over previous
import functools

import jax
import jax.numpy as jnp
from jax import lax
from jax.experimental import pallas as pl
from jax.experimental.pallas import tpu as pltpu
from jax.experimental.pallas import tpu_sc as plsc

_N = 20
_P = _N ** 3
_L = 16
_T = 5
_LINES = 400 // _T
_G = _LINES // _L
_SLAB = _P // _T
_INF = float("inf")


@functools.lru_cache(maxsize=1)
def _build_sc_kernel():
    mesh = plsc.VectorSubcoreMesh(
        core_axis_name="c", subcore_axis_name="s", num_cores=2, num_subcores=16
    )
    return functools.partial(
        pl.kernel,
        out_type=jax.ShapeDtypeStruct((4, _T, _L), jnp.float32),
        mesh=mesh,
        scratch_types=[
            pltpu.VMEM((_P,), jnp.float32),
            pltpu.VMEM((_P,), jnp.float32),
            pltpu.VMEM((_P,), jnp.float32),
            pltpu.VMEM((_P,), jnp.float32),
            pltpu.VMEM((_L,), jnp.float32),
            pltpu.VMEM((_L,), jnp.float32),
            pltpu.VMEM_SHARED((2 * _P,), jnp.float32),
            pltpu.SemaphoreType.DMA,
            pltpu.SemaphoreType.DMA,
        ],
        compiler_params=pltpu.CompilerParams(needs_layout_passes=False),
    )(_hausdorff_sc_body)


def _hausdorff_sc_body(pred_hbm, targ_hbm, out_hbm,
                       src_v, oth_v, da_v, db_v, acc_v, res_v, shared,
                       sem_s, sem_o):
    c = lax.axis_index("c")
    s = lax.axis_index("s")
    active = s < 2 * _T
    vloc = s // _T
    vol = c * 2 + vloc
    t = s % _T
    batch = vol // 2
    is_a = (vol % 2) == 0
    lane = lax.iota(jnp.int32, _L)
    w = [[float((z - zp) ** 2) for z in range(_N)] for zp in range(_N)]

    def run_pass(load_in, handle_out, base_of, stride):
        def group_body(g, carry):
            line = t * _LINES + g * _L + lane
            base = base_of(line)
            ins = [load_in(base + stride * zp) for zp in range(_N)]
            for z in range(_N):
                o = ins[0] + w[0][z]
                for zp in range(1, _N):
                    o = jnp.minimum(o, ins[zp] + w[zp][z])
                handle_out(base + stride * z, o)
            return carry
        lax.fori_loop(0, _G, group_body, 0)

    @pl.when(active)
    def _phase1():
        @pl.when(is_a)
        def _():
            pltpu.async_copy(targ_hbm.at[batch], src_v, sem_s)
            pltpu.async_copy(pred_hbm.at[batch], oth_v, sem_o)

        @pl.when(jnp.logical_not(is_a))
        def _():
            pltpu.async_copy(pred_hbm.at[batch], src_v, sem_s)
            pltpu.async_copy(targ_hbm.at[batch], oth_v, sem_o)

        pltpu.make_async_copy(pred_hbm.at[batch], src_v, sem_s).wait()

        def load_z(idx):
            return jnp.where(plsc.load_gather(src_v, [idx]) > 0.5, 0.0, _INF)

        run_pass(load_z,
                 lambda idx, o: plsc.store_scatter(da_v, [idx], o),
                 lambda l: l * _N, 1)

        run_pass(lambda idx: plsc.load_gather(da_v, [idx]),
                 lambda idx, o: plsc.store_scatter(db_v, [idx], o),
                 lambda l: (l // _N) * (_N * _N) + (l % _N), _N)

        pltpu.sync_copy(db_v.at[pl.ds(t * _SLAB, _SLAB)],
                        shared.at[pl.ds(vloc * _P + t * _SLAB, _SLAB)])

    plsc.subcore_barrier()

    @pl.when(active)
    def _phase2():
        pltpu.make_async_copy(pred_hbm.at[batch], oth_v, sem_o).wait()
        pltpu.sync_copy(shared.at[pl.ds(vloc * _P, _P)], da_v)
        acc_v[...] = jnp.full((_L,), -1.0, jnp.float32)

        def reduce_x(idx, o):
            qs = plsc.load_gather(src_v, [idx])
            qo = plsc.load_gather(oth_v, [idx])
            q = (qo > 0.5) & jnp.logical_not(qs > 0.5)
            acc_v[...] = jnp.maximum(acc_v[...], jnp.where(q, o, -1.0))

        run_pass(lambda idx: plsc.load_gather(da_v, [idx]),
                 reduce_x, lambda l: l, _N * _N)

        mx = jnp.max(acc_v[...])
        res_v[...] = jnp.where(lane == 0, mx, -1.0)
        pltpu.sync_copy(res_v, out_hbm.at[vol, t])


def kernel(predict, target):
    pred = predict.reshape(2, _P)
    targ = target.reshape(2, _P)
    out = _build_sc_kernel()(pred, targ)
    maxq = jnp.max(out, axis=(1, 2))
    d = jnp.sqrt(jnp.maximum(maxq, 0.0)) / jnp.float32(_N)
    dist_a = jnp.where(maxq[0::2] < 0.0, 0.0, d[0::2])
    dist_b = jnp.where(maxq[1::2] < 0.0, 0.0,
                       jnp.where(maxq[1::2] > 1e9, 999.0, d[1::2]))
    return jnp.mean(jnp.maximum(dist_a, dist_b))


# speedup vs baseline: 30.2046x; 1.1530x over previous
import functools

import jax
import jax.numpy as jnp
from jax import lax
from jax.experimental import pallas as pl
from jax.experimental.pallas import tpu as pltpu
from jax.experimental.pallas import tpu_sc as plsc

_N = 20
_P = _N ** 3
_L = 16
_T = 4
_LINES = 400 // _T
_G = -(-_LINES // _L)
_SLAB = _P // _T
_ACC = 4 * _P
_INF = float("inf")


@functools.lru_cache(maxsize=1)
def _build_sc_kernel():
    mesh = plsc.VectorSubcoreMesh(
        core_axis_name="c", subcore_axis_name="s", num_cores=1, num_subcores=16
    )
    return functools.partial(
        pl.kernel,
        out_type=jax.ShapeDtypeStruct((_L,), jnp.float32),
        mesh=mesh,
        scratch_types=[
            pltpu.VMEM((_P,), jnp.float32),
            pltpu.VMEM((_P,), jnp.float32),
            pltpu.VMEM((_P,), jnp.float32),
            pltpu.VMEM((_P,), jnp.float32),
            pltpu.VMEM((_L,), jnp.float32),
            pltpu.VMEM((16 * _L,), jnp.float32),
            pltpu.VMEM((_L,), jnp.float32),
            pltpu.VMEM_SHARED((_ACC + 16 * _L,), jnp.float32),
            pltpu.SemaphoreType.DMA,
            pltpu.SemaphoreType.DMA,
        ],
        compiler_params=pltpu.CompilerParams(needs_layout_passes=False),
    )(_hausdorff_sc_body)


def _hausdorff_sc_body(pred_hbm, targ_hbm, out_hbm,
                       src_v, oth_v, da_v, db_v, acc_v, part_v, res_v,
                       shared, sem_s, sem_o):
    s = lax.axis_index("s")
    vol = s // _T
    t = s % _T
    batch = vol // 2
    is_a = (vol % 2) == 0
    lane = lax.iota(jnp.int32, _L)
    w = [[float((z - zp) ** 2) for z in range(_N)] for zp in range(_N)]

    def run_pass(load_in, handle_out, base_of, stride):
        def group_body(g, carry):
            line = jnp.minimum(t * _LINES + g * _L + lane, 399)
            base = base_of(line)
            ins = [load_in(base + stride * zp) for zp in range(_N)]
            for z in range(_N):
                o = ins[0] + w[0][z]
                for zp in range(1, _N):
                    o = jnp.minimum(o, ins[zp] + w[zp][z])
                handle_out(base + stride * z, o)
            return carry
        lax.fori_loop(0, _G, group_body, 0)

    @pl.when(is_a)
    def _():
        pltpu.async_copy(targ_hbm.at[batch], src_v, sem_s)
        pltpu.async_copy(pred_hbm.at[batch], oth_v, sem_o)

    @pl.when(jnp.logical_not(is_a))
    def _():
        pltpu.async_copy(pred_hbm.at[batch], src_v, sem_s)
        pltpu.async_copy(targ_hbm.at[batch], oth_v, sem_o)

    pltpu.make_async_copy(pred_hbm.at[batch], src_v, sem_s).wait()

    def load_z(idx):
        return jnp.where(plsc.load_gather(src_v, [idx]) > 0.5, 0.0, _INF)

    run_pass(load_z,
             lambda idx, o: plsc.store_scatter(da_v, [idx], o),
             lambda l: l * _N, 1)

    run_pass(lambda idx: plsc.load_gather(da_v, [idx]),
             lambda idx, o: plsc.store_scatter(db_v, [idx], o),
             lambda l: (l // _N) * (_N * _N) + (l % _N), _N)

    pltpu.sync_copy(db_v.at[pl.ds(t * _SLAB, _SLAB)],
                    shared.at[pl.ds(vol * _P + t * _SLAB, _SLAB)])

    plsc.subcore_barrier()

    pltpu.make_async_copy(pred_hbm.at[batch], oth_v, sem_o).wait()
    pltpu.sync_copy(shared.at[pl.ds(vol * _P, _P)], da_v)
    acc_v[...] = jnp.full((_L,), -1.0, jnp.float32)

    def reduce_x(idx, o):
        qs = plsc.load_gather(src_v, [idx])
        qo = plsc.load_gather(oth_v, [idx])
        q = (qo > 0.5) & jnp.logical_not(qs > 0.5)
        acc_v[...] = jnp.maximum(acc_v[...], jnp.where(q, o, -1.0))

    run_pass(lambda idx: plsc.load_gather(da_v, [idx]),
             reduce_x, lambda l: l, _N * _N)

    pltpu.sync_copy(acc_v, shared.at[pl.ds(_ACC + _L * s, _L)])

    plsc.subcore_barrier()

    @pl.when(s == 0)
    def _combine():
        pltpu.sync_copy(shared.at[pl.ds(_ACC, 16 * _L)], part_v)
        mqs = []
        for v in range(4):
            m = part_v[pl.ds(_L * (_T * v), _L)]
            for k in range(1, _T):
                m = jnp.maximum(m, part_v[pl.ds(_L * (_T * v + k), _L)])
            mqs.append(jnp.max(m))
        mq = jnp.where(lane == 0, mqs[0],
                       jnp.where(lane == 1, mqs[1],
                                 jnp.where(lane == 2, mqs[2], mqs[3])))
        x = jnp.maximum(mq, 0.0)
        seed = plsc.bitcast(
            (lax.shift_right_logical(plsc.bitcast(x, jnp.int32), 1)
             + jnp.int32(0x1FBD1DF6)), jnp.float32)
        y = seed
        for _ in range(3):
            y = 0.5 * (y + x / y)
        d = y * jnp.float32(1.0 / _N)
        da = jnp.where(mq < 0.0, 0.0, jnp.where(mq > 1e9, _INF, d))
        db = jnp.where(mq < 0.0, 0.0, jnp.where(mq > 1e9, 999.0, d))
        dist = jnp.where(lane % 2 == 1, db, da)
        h0 = jnp.max(jnp.where(lane < 2, dist, -_INF))
        h1 = jnp.max(jnp.where((lane >= 2) & (lane < 4), dist, -_INF))
        res_v[...] = jnp.where(lane == 0, 0.5 * (h0 + h1), 0.0)
        pltpu.sync_copy(res_v, out_hbm)


def kernel(predict, target):
    pred = predict.reshape(2, _P)
    targ = target.reshape(2, _P)
    out = _build_sc_kernel()(pred, targ)
    return out[0]
